# Initial kernel scaffold; baseline (speedup 1.0000x reference)
import jax
import jax.numpy as jnp
from jax.experimental import pallas as pl


def kernel(batch_full_edge_attr, emb):
    raise NotImplementedError("write your pallas kernel here")


# speedup vs baseline: 32.7996x; 32.7996x over previous
import dataclasses
import functools

import jax
import jax.numpy as jnp
from jax.experimental import pallas as pl
from jax.experimental.pallas import tpu as pltpu
from jax.experimental.pallas import tpu_sc as plsc

B = 16
N = 512
M = N * N
C = 8
T = 17
L = 16
CH = 4096


def _lookup_body(table_v, idx_blk, out_blk):
    @pl.loop(0, CH, step=L)
    def _(i):
        iv = idx_blk[0, pl.ds(i, L)]
        for c in range(C):
            cvec = jnp.full((L,), c, dtype=jnp.int32)
            out_blk[0, c, pl.ds(i, L)] = plsc.load_gather(table_v, [cvec, iv])


@jax.jit
def _edge_encode(idx, embt):
    mesh = plsc.VectorSubcoreMesh(core_axis_name="c", subcore_axis_name="s")
    cp = pltpu.CompilerParams()
    if "needs_layout_passes" in pltpu.CompilerParams.__dataclass_fields__:
        cp = dataclasses.replace(cp, needs_layout_passes=False)

    @functools.partial(
        pl.kernel,
        out_type=jax.ShapeDtypeStruct((B, C, M), jnp.float32),
        mesh=mesh,
        scratch_types=[pltpu.VMEM((C, T), jnp.float32)],
        compiler_params=cp,
    )
    def k(embt_hbm, idx_hbm, out_hbm, table_v):
        pltpu.sync_copy(embt_hbm, table_v)
        pltpu.emit_pipeline(
            functools.partial(_lookup_body, table_v),
            grid=(B, M // CH),
            in_specs=[pl.BlockSpec((1, CH), index_map=lambda b, j: (b, j))],
            out_specs=[pl.BlockSpec((1, C, CH), index_map=lambda b, j: (b, 0, j))],
            core_axis_name=("c", "s"),
            dimension_semantics=(pltpu.PARALLEL, pltpu.PARALLEL),
        )(idx_hbm, out_hbm)

    return k(embt, idx)


def kernel(batch_full_edge_attr, emb):
    idx = batch_full_edge_attr.reshape(B, M).astype(jnp.int32)
    embt = emb.T.astype(jnp.float32)
    out = _edge_encode(idx, embt)
    return out.reshape(B, C, N, N)


# speedup vs baseline: 46.8593x; 1.4287x over previous
import dataclasses
import functools

import jax
import jax.numpy as jnp
from jax.experimental import pallas as pl
from jax.experimental.pallas import tpu as pltpu
from jax.experimental.pallas import tpu_sc as plsc

B = 16
N = 512
C = 8
T = 17
L = 16
BI = 8


def _lookup_body(table_v, idx_blk, out_blk):
    @pl.loop(0, BI)
    def _(r):
        @pl.loop(0, N, step=L)
        def _(l):
            iv = idx_blk[0, r, pl.ds(l, L)]
            for c in range(C):
                cvec = jnp.full((L,), c, dtype=jnp.int32)
                out_blk[0, c, r, pl.ds(l, L)] = plsc.load_gather(
                    table_v, [cvec, iv]
                )


@jax.jit
def _edge_encode(idx, embt):
    mesh = plsc.VectorSubcoreMesh(core_axis_name="c", subcore_axis_name="s")
    cp = pltpu.CompilerParams()
    if "needs_layout_passes" in pltpu.CompilerParams.__dataclass_fields__:
        cp = dataclasses.replace(cp, needs_layout_passes=False)

    @functools.partial(
        pl.kernel,
        out_type=jax.ShapeDtypeStruct((B, C, N, N), jnp.float32),
        mesh=mesh,
        scratch_types=[pltpu.VMEM((C, T), jnp.float32)],
        compiler_params=cp,
    )
    def k(embt_hbm, idx_hbm, out_hbm, table_v):
        pltpu.sync_copy(embt_hbm, table_v)
        pltpu.emit_pipeline(
            functools.partial(_lookup_body, table_v),
            grid=(B, N // BI),
            in_specs=[
                pl.BlockSpec((1, BI, N), index_map=lambda b, j: (b, j, 0))
            ],
            out_specs=[
                pl.BlockSpec((1, C, BI, N), index_map=lambda b, j: (b, 0, j, 0))
            ],
            core_axis_name=("c", "s"),
            dimension_semantics=(pltpu.PARALLEL, pltpu.PARALLEL),
        )(idx_hbm, out_hbm)

    return k(embt, idx)


def kernel(batch_full_edge_attr, emb):
    idx = batch_full_edge_attr.reshape(B, N, N).astype(jnp.int32)
    embt = emb.T.astype(jnp.float32)
    return _edge_encode(idx, embt)


# speedup vs baseline: 160.5000x; 3.4251x over previous
import dataclasses
import functools

import jax
import jax.numpy as jnp
from jax.experimental import pallas as pl
from jax.experimental.pallas import tpu as pltpu
from jax.experimental.pallas import tpu_sc as plsc

B = 16
N = 512
C = 8
T = 17
TP = 24
L = 16
BI = 8


def _lookup_body(tabs, idx_blk, out_blk):
    @pl.loop(0, BI)
    def _(r):
        @plsc.parallel_loop(0, N, step=L, unroll=4)
        def _(l):
            iv = idx_blk[0, r, pl.ds(l, L)]
            for c in range(C):
                out_blk[0, c, r, pl.ds(l, L)] = plsc.load_gather(tabs[c], [iv])


@jax.jit
def _edge_encode(idx, embt):
    mesh = plsc.VectorSubcoreMesh(core_axis_name="c", subcore_axis_name="s")
    cp = pltpu.CompilerParams()
    if "needs_layout_passes" in pltpu.CompilerParams.__dataclass_fields__:
        cp = dataclasses.replace(cp, needs_layout_passes=False)

    @functools.partial(
        pl.kernel,
        out_type=jax.ShapeDtypeStruct((B, C, N, N), jnp.float32),
        mesh=mesh,
        scratch_types=[pltpu.VMEM((TP,), jnp.float32) for _ in range(C)],
        compiler_params=cp,
    )
    def k(embt_hbm, idx_hbm, out_hbm, *tabs):
        for c in range(C):
            pltpu.sync_copy(embt_hbm.at[c], tabs[c])
        pltpu.emit_pipeline(
            functools.partial(_lookup_body, tabs),
            grid=(B, N // BI),
            in_specs=[
                pl.BlockSpec((1, BI, N), index_map=lambda b, j: (b, j, 0))
            ],
            out_specs=[
                pl.BlockSpec((1, C, BI, N), index_map=lambda b, j: (b, 0, j, 0))
            ],
            core_axis_name=("c", "s"),
            dimension_semantics=(pltpu.PARALLEL, pltpu.PARALLEL),
        )(idx_hbm, out_hbm)

    return k(embt, idx)


def kernel(batch_full_edge_attr, emb):
    idx = batch_full_edge_attr.reshape(B, N, N).astype(jnp.int32)
    embt = jnp.zeros((C, TP), jnp.float32).at[:, :T].set(emb.T)
    return _edge_encode(idx, embt)


# speedup vs baseline: 166.6511x; 1.0383x over previous
import dataclasses
import functools

import jax
import jax.numpy as jnp
from jax.experimental import pallas as pl
from jax.experimental.pallas import tpu as pltpu
from jax.experimental.pallas import tpu_sc as plsc

B = 16
N = 512
C = 8
T = 17
TP = 32
L = 16
BI = 8


def _lookup_body(tabs, idx_blk, out_blk):
    @pl.loop(0, BI)
    def _(r):
        @plsc.parallel_loop(0, N, step=L, unroll=4)
        def _(l):
            iv = idx_blk[0, r, pl.ds(l, L)]
            for c in range(C):
                out_blk[0, c, r, pl.ds(l, L)] = plsc.load_gather(tabs[c], [iv])


@jax.jit
def _edge_encode(idx, emb):
    mesh = plsc.VectorSubcoreMesh(core_axis_name="c", subcore_axis_name="s")
    cp = pltpu.CompilerParams()
    if "needs_layout_passes" in pltpu.CompilerParams.__dataclass_fields__:
        cp = dataclasses.replace(cp, needs_layout_passes=False)

    @functools.partial(
        pl.kernel,
        out_type=jax.ShapeDtypeStruct((B, C, N, N), jnp.float32),
        mesh=mesh,
        scratch_types=[pltpu.VMEM((T, C), jnp.float32)]
        + [pltpu.VMEM((TP,), jnp.float32) for _ in range(C)],
        compiler_params=cp,
    )
    def k(emb_hbm, idx_hbm, out_hbm, emb_v, *tabs):
        pltpu.sync_copy(emb_hbm, emb_v)
        t_lo = jax.lax.iota(jnp.int32, L)
        t_hi = jnp.minimum(t_lo + L, T - 1)
        for c in range(C):
            cvec = jnp.full((L,), c, dtype=jnp.int32)
            tabs[c][pl.ds(0, L)] = plsc.load_gather(emb_v, [t_lo, cvec])
            tabs[c][pl.ds(L, L)] = plsc.load_gather(emb_v, [t_hi, cvec])
        pltpu.emit_pipeline(
            functools.partial(_lookup_body, tabs),
            grid=(B, N // BI),
            in_specs=[
                pl.BlockSpec((1, BI, N), index_map=lambda b, j: (b, j, 0))
            ],
            out_specs=[
                pl.BlockSpec((1, C, BI, N), index_map=lambda b, j: (b, 0, j, 0))
            ],
            core_axis_name=("c", "s"),
            dimension_semantics=(pltpu.PARALLEL, pltpu.PARALLEL),
        )(idx_hbm, out_hbm)

    return k(emb, idx)


def kernel(batch_full_edge_attr, emb):
    idx = batch_full_edge_attr.reshape(B, N, N).astype(jnp.int32)
    return _edge_encode(idx, emb.astype(jnp.float32))
